# Initial kernel scaffold; baseline (speedup 1.0000x reference)
import jax
import jax.numpy as jnp
from jax.experimental import pallas as pl


def kernel(x, table, W1, b1, W2, b2, Wh, bh):
    raise NotImplementedError("write your pallas kernel here")


# speedup vs baseline: 8.6270x; 8.6270x over previous
import functools

import jax
import jax.numpy as jnp
from jax import lax
from jax.experimental import pallas as pl
from jax.experimental.pallas import tpu as pltpu
from jax.experimental.pallas import tpu_sc as plsc

B, S = 4096, 200
D_MODEL, HIDDEN, N_CLASSES = 64, 128, 1
NUM_CORES, NUM_SUBCORES, LANES = 2, 16, 16
NW = NUM_CORES * NUM_SUBCORES
ROWS_PER_TILE = B // NW


def _pool_body(x_hbm, table_hbm, out_hbm, idx_v, rows_v, out_v, sem):
    wid = lax.axis_index("s") * NUM_CORES + lax.axis_index("c")
    base = wid * ROWS_PER_TILE

    zero_acc = jnp.zeros((LANES,), jnp.float32)

    def row_body(i, carry):
        b = base + i
        pltpu.sync_copy(x_hbm.at[pl.ds(b * S, S)], idx_v)
        cp1 = pltpu.async_copy(
            table_hbm.at[idx_v.at[pl.ds(0, 128)]], rows_v.at[pl.ds(0, 128)], sem
        )
        cp2 = pltpu.async_copy(
            table_hbm.at[idx_v.at[pl.ds(128, S - 128)]],
            rows_v.at[pl.ds(128, S - 128)],
            sem,
        )
        cp1.wait()
        cp2.wait()

        def acc_body(t, accs):
            a0, a1, a2, a3 = accs
            return (
                a0 + rows_v[t, pl.ds(0, LANES)],
                a1 + rows_v[t, pl.ds(LANES, LANES)],
                a2 + rows_v[t, pl.ds(2 * LANES, LANES)],
                a3 + rows_v[t, pl.ds(3 * LANES, LANES)],
            )

        acc = lax.fori_loop(0, S, acc_body, (zero_acc,) * 4)
        for j in range(4):
            out_v[i, pl.ds(j * LANES, LANES)] = acc[j]
        return carry

    lax.fori_loop(0, ROWS_PER_TILE, row_body, 0)
    pltpu.sync_copy(out_v, out_hbm.at[pl.ds(base, ROWS_PER_TILE)])


_pool = functools.partial(
    pl.kernel,
    mesh=plsc.VectorSubcoreMesh(core_axis_name="c", subcore_axis_name="s"),
    out_type=jax.ShapeDtypeStruct((B, D_MODEL), jnp.float32),
    scratch_types=[
        pltpu.VMEM((S,), jnp.int32),
        pltpu.VMEM((S, D_MODEL), jnp.float32),
        pltpu.VMEM((ROWS_PER_TILE, D_MODEL), jnp.float32),
        pltpu.SemaphoreType.DMA,
    ],
    compiler_params=pltpu.CompilerParams(use_tc_tiling_on_sc=False),
)(_pool_body)


def _mlp_body(s_ref, x_ref, t0_ref, w1_ref, b1_ref, w2_ref, b2_ref,
              wh_ref, bh_ref, o_ref):
    zf = jnp.sum((x_ref[...] == 0).astype(jnp.float32), axis=1, keepdims=True)
    denom = jnp.maximum(jnp.float32(S) - zf, 1.0)
    pooled = (s_ref[...] - zf * t0_ref[...]) / denom
    h1 = jnp.dot(pooled, w1_ref[...], preferred_element_type=jnp.float32)
    h1 = jnp.maximum(h1 + b1_ref[...], 0.0)
    h2 = jnp.dot(h1, w2_ref[...], preferred_element_type=jnp.float32)
    h2 = jnp.maximum(h2 + b2_ref[...], 0.0)
    o_ref[...] = jnp.dot(h2, wh_ref[...], preferred_element_type=jnp.float32) + bh_ref[...]


_MLP_BLK = 1024


def _mlp(sums, x, table0, W1, b1, W2, b2, Wh, bh):
    grid = (B // _MLP_BLK,)
    return pl.pallas_call(
        _mlp_body,
        grid=grid,
        in_specs=[
            pl.BlockSpec((_MLP_BLK, D_MODEL), lambda i: (i, 0)),
            pl.BlockSpec((_MLP_BLK, S), lambda i: (i, 0)),
            pl.BlockSpec((1, D_MODEL), lambda i: (0, 0)),
            pl.BlockSpec((D_MODEL, HIDDEN), lambda i: (0, 0)),
            pl.BlockSpec((1, HIDDEN), lambda i: (0, 0)),
            pl.BlockSpec((HIDDEN, HIDDEN), lambda i: (0, 0)),
            pl.BlockSpec((1, HIDDEN), lambda i: (0, 0)),
            pl.BlockSpec((HIDDEN, N_CLASSES), lambda i: (0, 0)),
            pl.BlockSpec((1, N_CLASSES), lambda i: (0, 0)),
        ],
        out_specs=pl.BlockSpec((_MLP_BLK, N_CLASSES), lambda i: (i, 0)),
        out_shape=jax.ShapeDtypeStruct((B, N_CLASSES), jnp.float32),
    )(sums, x, table0, W1, b1.reshape(1, HIDDEN), W2, b2.reshape(1, HIDDEN),
      Wh, bh.reshape(1, N_CLASSES))


@jax.jit
def kernel(x, table, W1, b1, W2, b2, Wh, bh):
    sums = _pool(x.reshape(-1), table)
    return _mlp(sums, x, table[0:1], W1, b1, W2, b2, Wh, bh)


# speedup vs baseline: 16.3120x; 1.8908x over previous
import functools

import jax
import jax.numpy as jnp
from jax import lax
from jax.experimental import pallas as pl
from jax.experimental.pallas import tpu as pltpu
from jax.experimental.pallas import tpu_sc as plsc

B, S = 4096, 200
D_MODEL, HIDDEN, N_CLASSES = 64, 128, 1
NUM_CORES, NUM_SUBCORES, LANES = 2, 16, 16
NW = NUM_CORES * NUM_SUBCORES
ROWS_PER_TILE = B // NW


def _pool_body(x_hbm, table_hbm, out_hbm, idx_v, rows_a, rows_b, out_v,
               sem_a, sem_b):
    wid = lax.axis_index("s") * NUM_CORES + lax.axis_index("c")
    base = wid * ROWS_PER_TILE

    pltpu.sync_copy(x_hbm.at[pl.ds(base * S, ROWS_PER_TILE * S)], idx_v)

    def issue(row, buf, sem):
        off = row * S
        pltpu.async_copy(
            table_hbm.at[idx_v.at[pl.ds(off, 128)]], buf.at[pl.ds(0, 128)], sem
        )
        pltpu.async_copy(
            table_hbm.at[idx_v.at[pl.ds(off + 128, S - 128)]],
            buf.at[pl.ds(128, S - 128)],
            sem,
        )

    def drain(buf, sem):
        pltpu.make_async_copy(table_hbm.at[pl.ds(0, S)], buf, sem).wait()

    zero_acc = jnp.zeros((LANES,), jnp.float32)

    def accum(buf, i_out):
        def acc_body(t, accs):
            a0, a1, a2, a3 = accs
            r = [
                [buf[4 * t + k, pl.ds(j * LANES, LANES)] for j in range(4)]
                for k in range(4)
            ]
            a0 = a0 + ((r[0][0] + r[1][0]) + (r[2][0] + r[3][0]))
            a1 = a1 + ((r[0][1] + r[1][1]) + (r[2][1] + r[3][1]))
            a2 = a2 + ((r[0][2] + r[1][2]) + (r[2][2] + r[3][2]))
            a3 = a3 + ((r[0][3] + r[1][3]) + (r[2][3] + r[3][3]))
            return (a0, a1, a2, a3)

        acc = lax.fori_loop(0, S // 4, acc_body, (zero_acc,) * 4)
        for j in range(4):
            out_v[i_out, pl.ds(j * LANES, LANES)] = acc[j]

    last = ROWS_PER_TILE - 1
    issue(0, rows_a, sem_a)
    issue(1, rows_b, sem_b)

    def row_pair(g, carry):
        a = 2 * g
        b = 2 * g + 1
        drain(rows_a, sem_a)
        accum(rows_a, a)
        issue(jnp.minimum(a + 2, last), rows_a, sem_a)
        drain(rows_b, sem_b)
        accum(rows_b, b)
        issue(jnp.minimum(b + 2, last), rows_b, sem_b)
        return carry

    lax.fori_loop(0, ROWS_PER_TILE // 2, row_pair, 0)
    drain(rows_a, sem_a)
    drain(rows_b, sem_b)
    pltpu.sync_copy(out_v, out_hbm.at[pl.ds(base, ROWS_PER_TILE)])


_pool = functools.partial(
    pl.kernel,
    mesh=plsc.VectorSubcoreMesh(core_axis_name="c", subcore_axis_name="s"),
    out_type=jax.ShapeDtypeStruct((B, D_MODEL), jnp.float32),
    scratch_types=[
        pltpu.VMEM((ROWS_PER_TILE * S,), jnp.int32),
        pltpu.VMEM((S, D_MODEL), jnp.float32),
        pltpu.VMEM((S, D_MODEL), jnp.float32),
        pltpu.VMEM((ROWS_PER_TILE, D_MODEL), jnp.float32),
        pltpu.SemaphoreType.DMA,
        pltpu.SemaphoreType.DMA,
    ],
    compiler_params=pltpu.CompilerParams(use_tc_tiling_on_sc=False),
)(_pool_body)


def _mlp_body(s_ref, x_ref, t0_ref, w1_ref, b1_ref, w2_ref, b2_ref,
              wh_ref, bh_ref, o_ref):
    zf = jnp.sum((x_ref[...] == 0).astype(jnp.float32), axis=1, keepdims=True)
    denom = jnp.maximum(jnp.float32(S) - zf, 1.0)
    pooled = (s_ref[...] - zf * t0_ref[...]) / denom
    h1 = jnp.dot(pooled, w1_ref[...], preferred_element_type=jnp.float32)
    h1 = jnp.maximum(h1 + b1_ref[...], 0.0)
    h2 = jnp.dot(h1, w2_ref[...], preferred_element_type=jnp.float32)
    h2 = jnp.maximum(h2 + b2_ref[...], 0.0)
    o_ref[...] = jnp.dot(h2, wh_ref[...], preferred_element_type=jnp.float32) + bh_ref[...]


_MLP_BLK = 1024


def _mlp(sums, x, table0, W1, b1, W2, b2, Wh, bh):
    grid = (B // _MLP_BLK,)
    return pl.pallas_call(
        _mlp_body,
        grid=grid,
        in_specs=[
            pl.BlockSpec((_MLP_BLK, D_MODEL), lambda i: (i, 0)),
            pl.BlockSpec((_MLP_BLK, S), lambda i: (i, 0)),
            pl.BlockSpec((1, D_MODEL), lambda i: (0, 0)),
            pl.BlockSpec((D_MODEL, HIDDEN), lambda i: (0, 0)),
            pl.BlockSpec((1, HIDDEN), lambda i: (0, 0)),
            pl.BlockSpec((HIDDEN, HIDDEN), lambda i: (0, 0)),
            pl.BlockSpec((1, HIDDEN), lambda i: (0, 0)),
            pl.BlockSpec((HIDDEN, N_CLASSES), lambda i: (0, 0)),
            pl.BlockSpec((1, N_CLASSES), lambda i: (0, 0)),
        ],
        out_specs=pl.BlockSpec((_MLP_BLK, N_CLASSES), lambda i: (i, 0)),
        out_shape=jax.ShapeDtypeStruct((B, N_CLASSES), jnp.float32),
    )(sums, x, table0, W1, b1.reshape(1, HIDDEN), W2, b2.reshape(1, HIDDEN),
      Wh, bh.reshape(1, N_CLASSES))


@jax.jit
def kernel(x, table, W1, b1, W2, b2, Wh, bh):
    sums = _pool(x.reshape(-1), table)
    return _mlp(sums, x, table[0:1], W1, b1, W2, b2, Wh, bh)
